# Initial kernel scaffold; baseline (speedup 1.0000x reference)
import jax
import jax.numpy as jnp
from jax.experimental import pallas as pl


def kernel(input, adj, W, a):
    raise NotImplementedError("write your pallas kernel here")


# speedup vs baseline: 1721.2918x; 1721.2918x over previous
import jax
import jax.numpy as jnp
from jax import lax
from jax.experimental import pallas as pl
from jax.experimental.pallas import tpu as pltpu


def _gat_kernel(inp_ref, w_ref, a1_ref, a2_ref, adj_ref, out_ref, h_ref, g_ref):
    i = pl.program_id(0)

    @pl.when(i == 0)
    def _():
        h = jnp.dot(inp_ref[...], w_ref[...], preferred_element_type=jnp.float32)
        h_ref[...] = h
        g_ref[...] = lax.dot_general(
            a2_ref[...], h, (((1,), (1,)), ((), ())),
            preferred_element_type=jnp.float32)

    blk = out_ref.shape[0]
    h_blk = h_ref[pl.ds(i * blk, blk), :]
    f = lax.dot_general(
        h_blk, a1_ref[...], (((1,), (1,)), ((), ())),
        preferred_element_type=jnp.float32)
    s = f + g_ref[...]
    e = jnp.exp(-jnp.where(s >= 0.0, s, 0.01 * s))
    e = jnp.where(adj_ref[...] != 0, e, 0.0)
    rowsum = jnp.sum(e, axis=1, keepdims=True)
    hp = jnp.dot(e, h_ref[...], preferred_element_type=jnp.float32)
    hp = hp / rowsum
    out_ref[...] = jnp.where(hp > 0.0, hp, jnp.exp(hp) - 1.0)


def kernel(input, adj, W, a):
    n, d_in = input.shape
    d_out = W.shape[1]
    a1 = a[:, :d_out]
    a2 = a[:, d_out:]
    blk = 256
    return pl.pallas_call(
        _gat_kernel,
        grid=(n // blk,),
        in_specs=[
            pl.BlockSpec((n, d_in), lambda i: (0, 0)),
            pl.BlockSpec((d_in, d_out), lambda i: (0, 0)),
            pl.BlockSpec((1, d_out), lambda i: (0, 0)),
            pl.BlockSpec((1, d_out), lambda i: (0, 0)),
            pl.BlockSpec((blk, n), lambda i: (i, 0)),
        ],
        out_specs=pl.BlockSpec((blk, d_out), lambda i: (i, 0)),
        out_shape=jax.ShapeDtypeStruct((n, d_out), jnp.float32),
        scratch_shapes=[
            pltpu.VMEM((n, d_out), jnp.float32),
            pltpu.VMEM((1, n), jnp.float32),
        ],
    )(input, W, a1, a2, adj)


# speedup vs baseline: 1745.4305x; 1.0140x over previous
import jax
import jax.numpy as jnp
from jax import lax
from jax.experimental import pallas as pl
from jax.experimental.pallas import tpu as pltpu


def _gat_kernel(inp_ref, w_ref, a1_ref, a2_ref, adj_ref, out_ref, h_ref, f_ref, g_ref):
    i = pl.program_id(0)

    @pl.when(i == 0)
    def _():
        h = jnp.dot(inp_ref[...], w_ref[...], preferred_element_type=jnp.float32)
        h_ref[...] = h
        f_ref[...] = lax.dot_general(
            h, a1_ref[...], (((1,), (1,)), ((), ())),
            preferred_element_type=jnp.float32)
        g_ref[...] = lax.dot_general(
            a2_ref[...], h, (((1,), (1,)), ((), ())),
            preferred_element_type=jnp.float32)

    blk = out_ref.shape[0]
    f = f_ref[pl.ds(i * blk, blk), :]
    s = f + g_ref[...]
    e = jnp.exp(jnp.minimum(-s, -0.01 * s))
    e = jnp.where(adj_ref[...] != 0, e, 0.0)
    rowsum = jnp.sum(e, axis=1, keepdims=True)
    hp = jnp.dot(e, h_ref[...], preferred_element_type=jnp.float32)
    hp = hp / rowsum
    out_ref[...] = jnp.where(hp > 0.0, hp, jnp.exp(hp) - 1.0)


def kernel(input, adj, W, a):
    n, d_in = input.shape
    d_out = W.shape[1]
    a1 = a[:, :d_out]
    a2 = a[:, d_out:]
    blk = 256
    return pl.pallas_call(
        _gat_kernel,
        grid=(n // blk,),
        in_specs=[
            pl.BlockSpec((n, d_in), lambda i: (0, 0)),
            pl.BlockSpec((d_in, d_out), lambda i: (0, 0)),
            pl.BlockSpec((1, d_out), lambda i: (0, 0)),
            pl.BlockSpec((1, d_out), lambda i: (0, 0)),
            pl.BlockSpec((blk, n), lambda i: (i, 0)),
        ],
        out_specs=pl.BlockSpec((blk, d_out), lambda i: (i, 0)),
        out_shape=jax.ShapeDtypeStruct((n, d_out), jnp.float32),
        scratch_shapes=[
            pltpu.VMEM((n, d_out), jnp.float32),
            pltpu.VMEM((n, 1), jnp.float32),
            pltpu.VMEM((1, n), jnp.float32),
        ],
    )(input, W, a1, a2, adj)


# speedup vs baseline: 1979.4333x; 1.1341x over previous
import jax
import jax.numpy as jnp
from jax import lax
from jax.experimental import pallas as pl
from jax.experimental.pallas import tpu as pltpu


def _gat_kernel(inp_ref, w_ref, a1_ref, a2_ref, adj_ref, out_ref, h_ref, f_ref, g_ref):
    i = pl.program_id(0)

    @pl.when(i == 0)
    def _():
        h = jnp.dot(inp_ref[...], w_ref[...], preferred_element_type=jnp.float32)
        h_ref[...] = h
        f_ref[...] = lax.dot_general(
            h, a1_ref[...], (((1,), (1,)), ((), ())),
            preferred_element_type=jnp.float32)
        g_ref[...] = lax.dot_general(
            a2_ref[...], h, (((1,), (1,)), ((), ())),
            preferred_element_type=jnp.float32)

    blk = out_ref.shape[0]
    f = f_ref[pl.ds(i * blk, blk), :]
    s = f + g_ref[...]
    e = jnp.exp(jnp.minimum(-s, -0.01 * s))
    e = jnp.where(adj_ref[...] != 0, e, 0.0)
    rowsum = jnp.sum(e, axis=1, keepdims=True)
    hp = jnp.dot(e, h_ref[...], preferred_element_type=jnp.float32)
    hp = hp / rowsum
    out_ref[...] = jnp.where(hp > 0.0, hp, jnp.exp(hp) - 1.0)


def kernel(input, adj, W, a):
    n, d_in = input.shape
    d_out = W.shape[1]
    a1 = a[:, :d_out]
    a2 = a[:, d_out:]
    blk = 512
    return pl.pallas_call(
        _gat_kernel,
        grid=(n // blk,),
        in_specs=[
            pl.BlockSpec((n, d_in), lambda i: (0, 0)),
            pl.BlockSpec((d_in, d_out), lambda i: (0, 0)),
            pl.BlockSpec((1, d_out), lambda i: (0, 0)),
            pl.BlockSpec((1, d_out), lambda i: (0, 0)),
            pl.BlockSpec((blk, n), lambda i: (i, 0)),
        ],
        out_specs=pl.BlockSpec((blk, d_out), lambda i: (i, 0)),
        out_shape=jax.ShapeDtypeStruct((n, d_out), jnp.float32),
        scratch_shapes=[
            pltpu.VMEM((n, d_out), jnp.float32),
            pltpu.VMEM((n, 1), jnp.float32),
            pltpu.VMEM((1, n), jnp.float32),
        ],
    )(input, W, a1, a2, adj)
